# initial kernel scaffold (unmeasured)
import jax
import jax.numpy as jnp
from jax import lax
from jax.experimental import pallas as pl
from jax.experimental.pallas import tpu as pltpu

N_DEV = 8
B = 512
B_PER = B // N_DEV
D = 1024
H_PER = 2048
BF = jnp.bfloat16


def kernel(x, Win0, Wout0, Win1, Wout1, Win2, Wout2):
    def body(x_ref, win0_ref, wout0_ref, win1_ref, wout1_ref, win2_ref,
             wout2_ref, out_ref,
             xbufs, rsbufs, ps, wb, wob, send_sems, ag_sems, rs_sems):
        me = lax.axis_index("i")
        wins = [win0_ref, win1_ref, win2_ref]
        wouts = [wout0_ref, wout1_ref, wout2_ref]

        barrier = pltpu.get_barrier_semaphore()
        for j in range(1, N_DEV):
            pl.semaphore_signal(
                barrier, inc=1,
                device_id=(lax.rem(me + j, N_DEV),),
                device_id_type=pl.DeviceIdType.MESH,
            )
        pl.semaphore_wait(barrier, N_DEV - 1)

        def ag_phase(b):
            my_rows = xbufs.at[b, pl.ds(me * B_PER, B_PER), :]
            descs = []
            for j in range(1, N_DEV):
                dst = lax.rem(me + j, N_DEV)
                rdma = pltpu.make_async_remote_copy(
                    src_ref=my_rows,
                    dst_ref=my_rows,
                    send_sem=send_sems.at[j - 1],
                    recv_sem=ag_sems.at[b, j - 1],
                    device_id=(dst,),
                    device_id_type=pl.DeviceIdType.MESH,
                )
                rdma.start()
                descs.append(rdma)
            for d in descs:
                d.wait_recv()
            for d in descs:
                d.wait_send()

        def rs_phase(l, P):
            ps[...] = P.astype(BF)
            descs = []
            for j in range(1, N_DEV):
                dst = lax.rem(me + j, N_DEV)
                rdma = pltpu.make_async_remote_copy(
                    src_ref=ps.at[pl.ds(dst * B_PER, B_PER), :],
                    dst_ref=rsbufs.at[l, j - 1],
                    send_sem=send_sems.at[j - 1],
                    recv_sem=rs_sems.at[l, j - 1],
                    device_id=(dst,),
                    device_id_type=pl.DeviceIdType.MESH,
                )
                rdma.start()
                descs.append(rdma)
            acc = lax.dynamic_slice(P, (me * B_PER, 0), (B_PER, D))
            for d in descs:
                d.wait_recv()
            for j in range(1, N_DEV):
                acc = acc + rsbufs[l, j - 1, :, :].astype(jnp.float32)
            for d in descs:
                d.wait_send()
            return acc

        def layer(b, win_ref, wout_ref):
            wb[...] = win_ref[...].astype(BF)
            wob[...] = wout_ref[...].astype(BF)
            X = xbufs[b, :, :]
            h = jnp.dot(X, wb[...], preferred_element_type=jnp.float32)
            h = jnp.maximum(h, 0.0).astype(BF)
            return jnp.dot(h, wob[...], preferred_element_type=jnp.float32)

        xbufs[0, pl.ds(me * B_PER, B_PER), :] = x_ref[...].astype(BF)
        ag_phase(0)

        for l in range(3):
            P = layer(l, wins[l], wouts[l])
            Y = rs_phase(l, P)
            xbufs[l + 1, pl.ds(me * B_PER, B_PER), :] = Y.astype(BF)
            ag_phase(l + 1)

        out_ref[...] = xbufs[3, :, :].astype(jnp.float32)

    return pl.pallas_call(
        body,
        out_shape=jax.ShapeDtypeStruct((B, D), jnp.float32),
        in_specs=[pl.BlockSpec(memory_space=pltpu.VMEM)] * 7,
        out_specs=pl.BlockSpec(memory_space=pltpu.VMEM),
        scratch_shapes=[
            pltpu.VMEM((4, B, D), BF),
            pltpu.VMEM((3, N_DEV - 1, B_PER, D), BF),
            pltpu.VMEM((B, D), BF),
            pltpu.VMEM((D, H_PER), BF),
            pltpu.VMEM((H_PER, D), BF),
            pltpu.SemaphoreType.DMA((N_DEV - 1,)),
            pltpu.SemaphoreType.DMA((4, N_DEV - 1)),
            pltpu.SemaphoreType.DMA((3, N_DEV - 1)),
        ],
        compiler_params=pltpu.CompilerParams(collective_id=0),
    )(x, Win0, Wout0, Win1, Wout1, Win2, Wout2)


# baseline (device time: 99600 ns/iter reference)
import jax
import jax.numpy as jnp
from jax import lax
from jax.experimental import pallas as pl
from jax.experimental.pallas import tpu as pltpu

N_DEV = 8
B = 512
B_PER = B // N_DEV
D = 1024
H_PER = 2048
BF = jnp.bfloat16


def kernel(x, Win0, Wout0, Win1, Wout1, Win2, Wout2):
    def body(x_ref, win0_ref, wout0_ref, win1_ref, wout1_ref, win2_ref,
             wout2_ref, out_ref,
             xbufs, rsbufs, ps, pf, wb, wob, send_sems, ag_sems, rs_sems):
        me = lax.axis_index("i")
        wins = [win0_ref, win1_ref, win2_ref]
        wouts = [wout0_ref, wout1_ref, wout2_ref]

        barrier = pltpu.get_barrier_semaphore()
        for j in range(1, N_DEV):
            pl.semaphore_signal(
                barrier, inc=1,
                device_id=(lax.rem(me + j, N_DEV),),
                device_id_type=pl.DeviceIdType.MESH,
            )
        pl.semaphore_wait(barrier, N_DEV - 1)

        def ag_phase(b):
            my_rows = xbufs.at[b, pl.ds(me * B_PER, B_PER), :]
            descs = []
            for j in range(1, N_DEV):
                dst = lax.rem(me + j, N_DEV)
                rdma = pltpu.make_async_remote_copy(
                    src_ref=my_rows,
                    dst_ref=my_rows,
                    send_sem=send_sems.at[j - 1],
                    recv_sem=ag_sems.at[b, j - 1],
                    device_id=(dst,),
                    device_id_type=pl.DeviceIdType.MESH,
                )
                rdma.start()
                descs.append(rdma)
            for d in descs:
                d.wait_recv()
            for d in descs:
                d.wait_send()

        def rs_phase(l, P):
            pf[...] = P
            ps[...] = P.astype(BF)
            descs = []
            for j in range(1, N_DEV):
                dst = lax.rem(me + j, N_DEV)
                rdma = pltpu.make_async_remote_copy(
                    src_ref=ps.at[pl.ds(dst * B_PER, B_PER), :],
                    dst_ref=rsbufs.at[l, j - 1],
                    send_sem=send_sems.at[j - 1],
                    recv_sem=rs_sems.at[l, j - 1],
                    device_id=(dst,),
                    device_id_type=pl.DeviceIdType.MESH,
                )
                rdma.start()
                descs.append(rdma)
            acc = pf[pl.ds(me * B_PER, B_PER), :]
            for d in descs:
                d.wait_recv()
            for j in range(1, N_DEV):
                acc = acc + rsbufs[l, j - 1, :, :].astype(jnp.float32)
            for d in descs:
                d.wait_send()
            return acc

        def layer(b, win_ref, wout_ref):
            wb[...] = win_ref[...].astype(BF)
            wob[...] = wout_ref[...].astype(BF)
            X = xbufs[b, :, :]
            h = jnp.dot(X, wb[...], preferred_element_type=jnp.float32)
            h = jnp.maximum(h, 0.0).astype(BF)
            return jnp.dot(h, wob[...], preferred_element_type=jnp.float32)

        xbufs[0, pl.ds(me * B_PER, B_PER), :] = x_ref[...].astype(BF)
        ag_phase(0)

        for l in range(3):
            P = layer(l, wins[l], wouts[l])
            Y = rs_phase(l, P)
            xbufs[l + 1, pl.ds(me * B_PER, B_PER), :] = Y.astype(BF)
            ag_phase(l + 1)

        out_ref[...] = xbufs[3, :, :].astype(jnp.float32)

    return pl.pallas_call(
        body,
        out_shape=jax.ShapeDtypeStruct((B, D), jnp.float32),
        in_specs=[pl.BlockSpec(memory_space=pltpu.VMEM)] * 7,
        out_specs=pl.BlockSpec(memory_space=pltpu.VMEM),
        scratch_shapes=[
            pltpu.VMEM((4, B, D), BF),
            pltpu.VMEM((3, N_DEV - 1, B_PER, D), BF),
            pltpu.VMEM((B, D), BF),
            pltpu.VMEM((B, D), jnp.float32),
            pltpu.VMEM((D, H_PER), BF),
            pltpu.VMEM((H_PER, D), BF),
            pltpu.SemaphoreType.DMA((N_DEV - 1,)),
            pltpu.SemaphoreType.DMA((4, N_DEV - 1)),
            pltpu.SemaphoreType.DMA((3, N_DEV - 1)),
        ],
        compiler_params=pltpu.CompilerParams(
            collective_id=0, vmem_limit_bytes=100 * 1024 * 1024
        ),
    )(x, Win0, Wout0, Win1, Wout1, Win2, Wout2)


# device time: 88983 ns/iter; 1.1193x vs baseline; 1.1193x over previous
import jax
import jax.numpy as jnp
from jax import lax
from jax.experimental import pallas as pl
from jax.experimental.pallas import tpu as pltpu

N_DEV = 8
B = 512
B_PER = B // N_DEV
D = 1024
H_PER = 2048
BF = jnp.bfloat16
F32 = jnp.float32


def kernel(x, Win0, Wout0, Win1, Wout1, Win2, Wout2):
    def body(x_ref, win0_ref, wout0_ref, win1_ref, wout1_ref, win2_ref,
             wout2_ref, out_ref,
             xbufs, rsbufs, ps, wb, wob, win_stage, wout_stage,
             ag_send_sems, rs_send_sems, ag_sems, rs_sems, dma_sems):
        me = lax.axis_index("i")
        wins = [win0_ref, win1_ref, win2_ref]
        wouts = [wout0_ref, wout1_ref, wout2_ref]
        my_rows = pl.ds(me * B_PER, B_PER)

        barrier = pltpu.get_barrier_semaphore()
        for j in range(1, N_DEV):
            pl.semaphore_signal(
                barrier, inc=1,
                device_id=(lax.rem(me + j, N_DEV),),
                device_id_type=pl.DeviceIdType.MESH,
            )
        pl.semaphore_wait(barrier, N_DEV - 1)

        def ag_send(b):
            src = xbufs.at[b, my_rows, :]
            descs = []
            for j in range(1, N_DEV):
                rdma = pltpu.make_async_remote_copy(
                    src_ref=src,
                    dst_ref=src,
                    send_sem=ag_send_sems.at[j - 1],
                    recv_sem=ag_sems.at[b, j - 1],
                    device_id=(lax.rem(me + j, N_DEV),),
                    device_id_type=pl.DeviceIdType.MESH,
                )
                rdma.start()
                descs.append(rdma)
            return descs

        def block_partial(b, rows):
            xblk = xbufs[b, rows, :]
            h = jnp.dot(xblk, wb[...], preferred_element_type=F32)
            h = jnp.maximum(h, 0.0).astype(BF)
            return jnp.dot(h, wob[...], preferred_element_type=F32)

        xbufs[0, my_rows, :] = x_ref[...].astype(BF)
        ag_descs = [None] * 4
        ag_descs[0] = ag_send(0)

        prev_rs_descs = None
        Y = None
        for l in range(3):
            cp_in = pltpu.make_async_copy(wins[l], win_stage, dma_sems.at[0])
            cp_out = pltpu.make_async_copy(wouts[l], wout_stage, dma_sems.at[1])
            cp_in.start()
            cp_out.start()
            cp_in.wait()
            wb[...] = win_stage[...].astype(BF)
            cp_out.wait()
            wob[...] = wout_stage[...].astype(BF)

            acc = block_partial(l, my_rows)

            rs_descs = []
            for j in range(1, N_DEV):
                ag_descs[l][j - 1].wait_recv()
                s = lax.rem(me - j + N_DEV, N_DEV)
                s_rows = pl.ds(s * B_PER, B_PER)
                p = block_partial(l, s_rows)
                ps[s_rows, :] = p.astype(BF)
                if prev_rs_descs is not None:
                    prev_rs_descs[j - 1].wait_send()
                rdma = pltpu.make_async_remote_copy(
                    src_ref=ps.at[s_rows, :],
                    dst_ref=rsbufs.at[l, 7 - j],
                    send_sem=rs_send_sems.at[j - 1],
                    recv_sem=rs_sems.at[l, 7 - j],
                    device_id=(s,),
                    device_id_type=pl.DeviceIdType.MESH,
                )
                rdma.start()
                rs_descs.append(rdma)

            for d in rs_descs:
                d.wait_recv()
            Y = acc
            for k in range(N_DEV - 1):
                Y = Y + rsbufs[l, k, :, :].astype(F32)

            xbufs[l + 1, my_rows, :] = Y.astype(BF)
            for d in ag_descs[l]:
                d.wait_send()
            ag_descs[l + 1] = ag_send(l + 1)
            prev_rs_descs = rs_descs

        for d in ag_descs[3]:
            d.wait_recv()
        out_ref[...] = xbufs[3, :, :].astype(F32)
        out_ref[my_rows, :] = Y

        for d in prev_rs_descs:
            d.wait_send()
        for d in ag_descs[3]:
            d.wait_send()

    return pl.pallas_call(
        body,
        out_shape=jax.ShapeDtypeStruct((B, D), F32),
        in_specs=[pl.BlockSpec(memory_space=pltpu.VMEM)]
        + [pl.BlockSpec(memory_space=pl.ANY)] * 6,
        out_specs=pl.BlockSpec(memory_space=pltpu.VMEM),
        scratch_shapes=[
            pltpu.VMEM((4, B, D), BF),
            pltpu.VMEM((3, N_DEV - 1, B_PER, D), BF),
            pltpu.VMEM((B, D), BF),
            pltpu.VMEM((D, H_PER), BF),
            pltpu.VMEM((H_PER, D), BF),
            pltpu.VMEM((D, H_PER), F32),
            pltpu.VMEM((H_PER, D), F32),
            pltpu.SemaphoreType.DMA((N_DEV - 1,)),
            pltpu.SemaphoreType.DMA((N_DEV - 1,)),
            pltpu.SemaphoreType.DMA((4, N_DEV - 1)),
            pltpu.SemaphoreType.DMA((3, N_DEV - 1)),
            pltpu.SemaphoreType.DMA((2,)),
        ],
        compiler_params=pltpu.CompilerParams(
            collective_id=0, vmem_limit_bytes=100 * 1024 * 1024
        ),
    )(x, Win0, Wout0, Win1, Wout1, Win2, Wout2)
